# baseline (speedup 1.0000x reference)
import functools

import jax
import jax.numpy as jnp
from jax import lax
from jax.experimental import pallas as pl
from jax.experimental.pallas import tpu as pltpu, tpu_sc as plsc

B = 16384
R = 16
G = 16
GW = G * R
NC = 2
NS = 16
NW = NC * NS
B_PER_W = B // NW
D0 = 1000000
D1 = 100000

W0_GPAD = 62504
W1_GPAD = 6256
W0_ALN = 999936
W1_ALN = 99968
W0_GPW = 2016
W0_WIN = 112
W0_NWIN = W0_GPW // W0_WIN
BUF_W = W0_WIN * G

CHUNK = 128
N_CHUNKS = B_PER_W // CHUNK


def _make_repack():
    mesh = plsc.VectorSubcoreMesh(core_axis_name="c", subcore_axis_name="s")

    @functools.partial(
        pl.kernel,
        mesh=mesh,
        compiler_params=pltpu.CompilerParams(needs_layout_passes=False),
        out_type=(
            jax.ShapeDtypeStruct((W0_GPAD, GW), jnp.float32),
            jax.ShapeDtypeStruct((W1_GPAD, GW), jnp.float32),
            jax.ShapeDtypeStruct((W1_GPAD, GW), jnp.float32),
        ),
        scratch_types=[
            pltpu.VMEM((R, BUF_W), jnp.float32),
            pltpu.VMEM((R, BUF_W), jnp.float32),
            pltpu.VMEM((W0_WIN, GW), jnp.float32),
            pltpu.VMEM((W0_WIN, GW), jnp.float32),
            pltpu.SemaphoreType.DMA,
            pltpu.SemaphoreType.DMA,
        ],
    )
    def ka(wt0_hbm, wt1_hbm, wt2_hbm, t0g_hbm, t1g_hbm, t2g_hbm,
           w0g_hbm, w1g_hbm, w2g_hbm, buf_a, buf_b, ob_a, ob_b,
           sem_r, sem_w):
        wid = lax.axis_index("s") * NC + lax.axis_index("c")
        bufs = (buf_a, buf_b)
        obs = (ob_a, ob_b)

        def repack(buf, ob, ngrp):
            def gl_body(g4, carry):
                for k in range(4):
                    gl = g4 * 4 + k
                    for r in range(R):
                        ob[gl, pl.ds(r * G, G)] = buf[r, pl.ds(gl * G, G)]
                return carry
            lax.fori_loop(0, ngrp // 4, gl_body, 0)

        def do_windows(wt_hbm, wg_hbm, windows, pending_w):
            def aln(x):
                return pl.multiple_of(x, 128)

            reads = [pltpu.async_copy(
                wt_hbm.at[:, pl.ds(aln(windows[0][0]), windows[0][1])],
                bufs[0].at[:, pl.ds(0, windows[0][1])], sem_r)]
            for u, (x0, width) in enumerate(windows):
                pb = u % 2
                if u + 1 < len(windows):
                    nx0, nw_ = windows[u + 1]
                    nxt = [pltpu.async_copy(
                        wt_hbm.at[:, pl.ds(aln(nx0), nw_)],
                        bufs[(u + 1) % 2].at[:, pl.ds(0, nw_)], sem_r)]
                else:
                    nxt = []
                for c in reads:
                    c.wait()
                if pending_w[pb] is not None:
                    pending_w[pb].wait()
                    pending_w[pb] = None
                ngrp = width // G
                repack(bufs[pb], obs[pb], ngrp)
                pending_w[pb] = pltpu.async_copy(
                    obs[pb].at[pl.ds(0, ngrp)],
                    wg_hbm.at[pl.ds(pl.multiple_of(x0 // G, 8), ngrp)], sem_w)
                reads = nxt
            return pending_w

        pending = [None, None]

        @pl.when(wid < NW - 1)
        def _():
            pend = [None, None]
            base = wid * (W0_GPW * G)
            win_list = [(base + u * BUF_W, BUF_W) for u in range(W0_NWIN)]
            pend = do_windows(wt0_hbm, w0g_hbm, win_list, pend)
            for c in pend:
                if c is not None:
                    c.wait()

        @pl.when(wid == NW - 1)
        def _():
            pltpu.sync_copy(t0g_hbm, ob_a.at[pl.ds(0, 4)])
            zero = jnp.zeros((R,), jnp.float32)

            def z_body(i, carry):
                ob_a[4 + (i // G), pl.ds((i % G) * R, R)] = zero
                return carry

            lax.fori_loop(0, 4 * (GW // R), z_body, 0)
            pltpu.sync_copy(ob_a.at[pl.ds(0, 8)],
                            w0g_hbm.at[pl.ds(W0_ALN // G, 8)])

        def small_table(wt_hbm, wg_hbm, tg_hbm, tail_wid):
            @pl.when(wid < 13)
            def _():
                pend = [None, None]
                base = wid * 200 * G
                wins = [(base, BUF_W), (base + BUF_W, 1408)]
                pend = do_windows(wt_hbm, wg_hbm, wins, pend)
                for c in pend:
                    if c is not None:
                        c.wait()

            @pl.when(wid >= 13)
            def _():
                pend = [None, None]
                base = (2600 + (wid - 13) * 192) * G
                wins = [(base, BUF_W), (base + BUF_W, 1280)]
                pend = do_windows(wt_hbm, wg_hbm, wins, pend)
                for c in pend:
                    if c is not None:
                        c.wait()

            @pl.when(wid == tail_wid)
            def _():
                pltpu.sync_copy(tg_hbm, ob_a.at[pl.ds(0, 2)])
                zero = jnp.zeros((R,), jnp.float32)

                def z_body(i, carry):
                    ob_a[2 + (i // G), pl.ds((i % G) * R, R)] = zero
                    return carry

                lax.fori_loop(0, 6 * (GW // R), z_body, 0)
                pltpu.sync_copy(ob_a.at[pl.ds(0, 8)],
                                wg_hbm.at[pl.ds(W1_ALN // G, 8)])

        small_table(wt1_hbm, w1g_hbm, t1g_hbm, NW - 2)
        small_table(wt2_hbm, w2g_hbm, t2g_hbm, NW - 3)

    return ka


def _make_gather():
    mesh = plsc.VectorSubcoreMesh(core_axis_name="c", subcore_axis_name="s")

    @functools.partial(
        pl.kernel,
        mesh=mesh,
        compiler_params=pltpu.CompilerParams(needs_layout_passes=False),
        out_type=jax.ShapeDtypeStruct((B,), jnp.float32),
        scratch_types=[
            pltpu.VMEM((B_PER_W,), jnp.int32),
            pltpu.VMEM((B_PER_W,), jnp.int32),
            pltpu.VMEM((B_PER_W,), jnp.int32),
            pltpu.VMEM((CHUNK,), jnp.int32),
            pltpu.VMEM((CHUNK,), jnp.int32),
            pltpu.VMEM((CHUNK,), jnp.int32),
            pltpu.VMEM((CHUNK, GW), jnp.float32),
            pltpu.VMEM((CHUNK, GW), jnp.float32),
            pltpu.VMEM((CHUNK, GW), jnp.float32),
            pltpu.VMEM((B_PER_W,), jnp.float32),
            pltpu.SemaphoreType.DMA,
        ],
    )
    def kb(i0_hbm, i1_hbm, i2_hbm, w0g_hbm, w1g_hbm, w2g_hbm, out_hbm,
           idx0_v, idx1_v, idx2_v, g0_v, g1_v, g2_v,
           grp0_v, grp1_v, grp2_v, out_v, sem):
        wid = lax.axis_index("s") * NC + lax.axis_index("c")
        base = wid * B_PER_W
        pltpu.sync_copy(i0_hbm.at[pl.ds(base, B_PER_W)], idx0_v)
        pltpu.sync_copy(i1_hbm.at[pl.ds(base, B_PER_W)], idx1_v)
        pltpu.sync_copy(i2_hbm.at[pl.ds(base, B_PER_W)], idx2_v)
        iota = lax.iota(jnp.int32, R)

        def chunk_body(c, carry):
            cbase = c * CHUNK

            def split_body(t, carry):
                o = cbase + t * R
                g0_v[pl.ds(t * R, R)] = lax.shift_right_logical(
                    idx0_v[pl.ds(o, R)], 4)
                g1_v[pl.ds(t * R, R)] = lax.shift_right_logical(
                    idx1_v[pl.ds(o, R)], 4)
                g2_v[pl.ds(t * R, R)] = lax.shift_right_logical(
                    idx2_v[pl.ds(o, R)], 4)
                return carry

            lax.fori_loop(0, CHUNK // R, split_body, 0)
            c0 = pltpu.async_copy(w0g_hbm.at[g0_v], grp0_v, sem)
            c1 = pltpu.async_copy(w1g_hbm.at[g1_v], grp1_v, sem)
            c2 = pltpu.async_copy(w2g_hbm.at[g2_v], grp2_v, sem)
            c0.wait()
            c1.wait()
            c2.wait()

            def body(t, carry):
                o = cbase + t * R
                row = t * R + iota
                e0 = idx0_v[pl.ds(o, R)] & (G - 1)
                e1 = idx1_v[pl.ds(o, R)] & (G - 1)
                e2 = idx2_v[pl.ds(o, R)] & (G - 1)
                acc = (plsc.load_gather(grp0_v, [row, e0])
                       * plsc.load_gather(grp1_v, [row, e1])
                       * plsc.load_gather(grp2_v, [row, e2]))
                for r in range(1, R):
                    acc = acc + (
                        plsc.load_gather(grp0_v, [row, e0 + r * G])
                        * plsc.load_gather(grp1_v, [row, e1 + r * G])
                        * plsc.load_gather(grp2_v, [row, e2 + r * G]))
                out_v[pl.ds(o, R)] = acc
                return carry

            lax.fori_loop(0, CHUNK // R, body, 0)
            return carry

        lax.fori_loop(0, N_CHUNKS, chunk_body, 0)
        pltpu.sync_copy(out_v, out_hbm.at[pl.ds(base, B_PER_W)])

    return kb


_repack = _make_repack()
_gather = _make_gather()


@jax.jit
def kernel(i0, i1, i2, W0, W1, W2):
    def tail_groups(w, aln):
        t = w[aln:]
        n = t.shape[0] // G
        return t.reshape(n, G, R).transpose(0, 2, 1).reshape(n, GW)

    t0g = tail_groups(W0, W0_ALN)
    t1g = tail_groups(W1, W1_ALN)
    t2g = tail_groups(W2, W1_ALN)
    w0g, w1g, w2g = _repack(W0.T, W1.T, W2.T, t0g, t1g, t2g)
    return _gather(i0, i1, i2, w0g, w1g, w2g)


# speedup vs baseline: 1.7985x; 1.7985x over previous
import functools

import jax
import jax.numpy as jnp
from jax import lax
from jax.experimental import pallas as pl
from jax.experimental.pallas import tpu as pltpu, tpu_sc as plsc

B = 16384
R = 16
G = 16
GW = G * R
NC = 2
NS = 16
NW = NC * NS
B_PER_W = B // NW
D0 = 1000000
D1 = 100000

W0_GPAD = 62504
W1_GPAD = 6256
W0_ALN = 999936
W1_ALN = 99968
W0_GPW = 2016
W0_WIN = 112
W0_NWIN = W0_GPW // W0_WIN
BUF_W = W0_WIN * G

CHUNK = 128
N_CHUNKS = B_PER_W // CHUNK


def _make_repack():
    mesh = plsc.VectorSubcoreMesh(core_axis_name="c", subcore_axis_name="s")

    @functools.partial(
        pl.kernel,
        mesh=mesh,
        compiler_params=pltpu.CompilerParams(needs_layout_passes=False),
        out_type=(
            jax.ShapeDtypeStruct((W0_GPAD, GW), jnp.float32),
            jax.ShapeDtypeStruct((W1_GPAD, GW), jnp.float32),
            jax.ShapeDtypeStruct((W1_GPAD, GW), jnp.float32),
        ),
        scratch_types=[
            pltpu.VMEM((R, BUF_W), jnp.float32),
            pltpu.VMEM((R, BUF_W), jnp.float32),
            pltpu.VMEM((W0_WIN, GW), jnp.float32),
            pltpu.VMEM((W0_WIN, GW), jnp.float32),
            pltpu.SemaphoreType.DMA,
            pltpu.SemaphoreType.DMA,
        ],
    )
    def ka(wt0_hbm, wt1_hbm, wt2_hbm, t0g_hbm, t1g_hbm, t2g_hbm,
           w0g_hbm, w1g_hbm, w2g_hbm, buf_a, buf_b, ob_a, ob_b,
           sem_r, sem_w):
        wid = lax.axis_index("s") * NC + lax.axis_index("c")
        bufs = (buf_a, buf_b)
        obs = (ob_a, ob_b)

        def repack(buf, ob, ngrp):
            @plsc.parallel_loop(0, ngrp, 1, unroll=2)
            def _(gl):
                for r in range(R):
                    ob[gl, pl.ds(r * G, G)] = buf[r, pl.ds(gl * G, G)]

        def do_windows(wt_hbm, wg_hbm, windows, pending_w):
            def aln(x):
                return pl.multiple_of(x, 128)

            reads = [pltpu.async_copy(
                wt_hbm.at[:, pl.ds(aln(windows[0][0]), windows[0][1])],
                bufs[0].at[:, pl.ds(0, windows[0][1])], sem_r)]
            for u, (x0, width) in enumerate(windows):
                pb = u % 2
                if u + 1 < len(windows):
                    nx0, nw_ = windows[u + 1]
                    nxt = [pltpu.async_copy(
                        wt_hbm.at[:, pl.ds(aln(nx0), nw_)],
                        bufs[(u + 1) % 2].at[:, pl.ds(0, nw_)], sem_r)]
                else:
                    nxt = []
                for c in reads:
                    c.wait()
                if pending_w[pb] is not None:
                    pending_w[pb].wait()
                    pending_w[pb] = None
                ngrp = width // G
                repack(bufs[pb], obs[pb], ngrp)
                pending_w[pb] = pltpu.async_copy(
                    obs[pb].at[pl.ds(0, ngrp)],
                    wg_hbm.at[pl.ds(pl.multiple_of(x0 // G, 8), ngrp)], sem_w)
                reads = nxt
            return pending_w

        pending = [None, None]

        @pl.when(wid < NW - 1)
        def _():
            pend = [None, None]
            base = wid * (W0_GPW * G)
            win_list = [(base + u * BUF_W, BUF_W) for u in range(W0_NWIN)]
            pend = do_windows(wt0_hbm, w0g_hbm, win_list, pend)
            for c in pend:
                if c is not None:
                    c.wait()

        @pl.when(wid == NW - 1)
        def _():
            pltpu.sync_copy(t0g_hbm, ob_a.at[pl.ds(0, 4)])
            zero = jnp.zeros((R,), jnp.float32)

            def z_body(i, carry):
                ob_a[4 + (i // G), pl.ds((i % G) * R, R)] = zero
                return carry

            lax.fori_loop(0, 4 * (GW // R), z_body, 0)
            pltpu.sync_copy(ob_a.at[pl.ds(0, 8)],
                            w0g_hbm.at[pl.ds(W0_ALN // G, 8)])

        def small_table(wt_hbm, wg_hbm, tg_hbm, tail_wid):
            @pl.when(wid < 13)
            def _():
                pend = [None, None]
                base = wid * 200 * G
                wins = [(base, BUF_W), (base + BUF_W, 1408)]
                pend = do_windows(wt_hbm, wg_hbm, wins, pend)
                for c in pend:
                    if c is not None:
                        c.wait()

            @pl.when(wid >= 13)
            def _():
                pend = [None, None]
                base = (2600 + (wid - 13) * 192) * G
                wins = [(base, BUF_W), (base + BUF_W, 1280)]
                pend = do_windows(wt_hbm, wg_hbm, wins, pend)
                for c in pend:
                    if c is not None:
                        c.wait()

            @pl.when(wid == tail_wid)
            def _():
                pltpu.sync_copy(tg_hbm, ob_a.at[pl.ds(0, 2)])
                zero = jnp.zeros((R,), jnp.float32)

                def z_body(i, carry):
                    ob_a[2 + (i // G), pl.ds((i % G) * R, R)] = zero
                    return carry

                lax.fori_loop(0, 6 * (GW // R), z_body, 0)
                pltpu.sync_copy(ob_a.at[pl.ds(0, 8)],
                                wg_hbm.at[pl.ds(W1_ALN // G, 8)])

        small_table(wt1_hbm, w1g_hbm, t1g_hbm, NW - 2)
        small_table(wt2_hbm, w2g_hbm, t2g_hbm, NW - 3)

    return ka


def _make_gather():
    mesh = plsc.VectorSubcoreMesh(core_axis_name="c", subcore_axis_name="s")

    @functools.partial(
        pl.kernel,
        mesh=mesh,
        compiler_params=pltpu.CompilerParams(needs_layout_passes=False),
        out_type=jax.ShapeDtypeStruct((B,), jnp.float32),
        scratch_types=[
            pltpu.VMEM((B_PER_W,), jnp.int32),
            pltpu.VMEM((B_PER_W,), jnp.int32),
            pltpu.VMEM((B_PER_W,), jnp.int32),
            pltpu.VMEM((CHUNK,), jnp.int32),
            pltpu.VMEM((CHUNK,), jnp.int32),
            pltpu.VMEM((CHUNK,), jnp.int32),
            pltpu.VMEM((CHUNK, GW), jnp.float32),
            pltpu.VMEM((CHUNK, GW), jnp.float32),
            pltpu.VMEM((CHUNK, GW), jnp.float32),
            pltpu.VMEM((B_PER_W,), jnp.float32),
            pltpu.SemaphoreType.DMA,
        ],
    )
    def kb(i0_hbm, i1_hbm, i2_hbm, w0g_hbm, w1g_hbm, w2g_hbm, out_hbm,
           idx0_v, idx1_v, idx2_v, g0_v, g1_v, g2_v,
           grp0_v, grp1_v, grp2_v, out_v, sem):
        wid = lax.axis_index("s") * NC + lax.axis_index("c")
        base = wid * B_PER_W
        pltpu.sync_copy(i0_hbm.at[pl.ds(base, B_PER_W)], idx0_v)
        pltpu.sync_copy(i1_hbm.at[pl.ds(base, B_PER_W)], idx1_v)
        pltpu.sync_copy(i2_hbm.at[pl.ds(base, B_PER_W)], idx2_v)
        iota = lax.iota(jnp.int32, R)

        def chunk_body(c, carry):
            cbase = c * CHUNK

            def split_body(t, carry):
                o = cbase + t * R
                g0_v[pl.ds(t * R, R)] = lax.shift_right_logical(
                    idx0_v[pl.ds(o, R)], 4)
                g1_v[pl.ds(t * R, R)] = lax.shift_right_logical(
                    idx1_v[pl.ds(o, R)], 4)
                g2_v[pl.ds(t * R, R)] = lax.shift_right_logical(
                    idx2_v[pl.ds(o, R)], 4)
                return carry

            lax.fori_loop(0, CHUNK // R, split_body, 0)
            c0 = pltpu.async_copy(w0g_hbm.at[g0_v], grp0_v, sem)
            c1 = pltpu.async_copy(w1g_hbm.at[g1_v], grp1_v, sem)
            c2 = pltpu.async_copy(w2g_hbm.at[g2_v], grp2_v, sem)
            c0.wait()
            c1.wait()
            c2.wait()

            def body(t, carry):
                o = cbase + t * R
                row = t * R + iota
                e0 = idx0_v[pl.ds(o, R)] & (G - 1)
                e1 = idx1_v[pl.ds(o, R)] & (G - 1)
                e2 = idx2_v[pl.ds(o, R)] & (G - 1)
                acc = (plsc.load_gather(grp0_v, [row, e0])
                       * plsc.load_gather(grp1_v, [row, e1])
                       * plsc.load_gather(grp2_v, [row, e2]))
                for r in range(1, R):
                    acc = acc + (
                        plsc.load_gather(grp0_v, [row, e0 + r * G])
                        * plsc.load_gather(grp1_v, [row, e1 + r * G])
                        * plsc.load_gather(grp2_v, [row, e2 + r * G]))
                out_v[pl.ds(o, R)] = acc
                return carry

            lax.fori_loop(0, CHUNK // R, body, 0)
            return carry

        lax.fori_loop(0, N_CHUNKS, chunk_body, 0)
        pltpu.sync_copy(out_v, out_hbm.at[pl.ds(base, B_PER_W)])

    return kb


_repack = _make_repack()
_gather = _make_gather()


@jax.jit
def kernel(i0, i1, i2, W0, W1, W2):
    def tail_groups(w, aln):
        t = w[aln:]
        n = t.shape[0] // G
        return t.reshape(n, G, R).transpose(0, 2, 1).reshape(n, GW)

    t0g = tail_groups(W0, W0_ALN)
    t1g = tail_groups(W1, W1_ALN)
    t2g = tail_groups(W2, W1_ALN)
    w0g, w1g, w2g = _repack(W0.T, W1.T, W2.T, t0g, t1g, t2g)
    return _gather(i0, i1, i2, w0g, w1g, w2g)
